# baseline (speedup 1.0000x reference)
import jax
import jax.numpy as jnp
from jax.experimental import pallas as pl
from jax.experimental.pallas import tpu as pltpu


_BM = 200


def _gcn_kernel(adj_ref, x_ref, w_ref, b_ref, out_ref, h_ref):
    @pl.when(pl.program_id(0) == 0)
    def _compute_h():
        h_ref[...] = jnp.dot(
            x_ref[...], w_ref[...], preferred_element_type=jnp.float32
        )

    out_ref[...] = (
        jnp.dot(adj_ref[...], h_ref[...], preferred_element_type=jnp.float32)
        + b_ref[...]
    )


def kernel(x, adj, W, b):
    n, d_in = x.shape
    d_out = W.shape[1]
    out = pl.pallas_call(
        _gcn_kernel,
        grid=(n // _BM,),
        in_specs=[
            pl.BlockSpec((_BM, n), lambda i: (i, 0)),
            pl.BlockSpec((n, d_in), lambda i: (0, 0)),
            pl.BlockSpec((d_in, d_out), lambda i: (0, 0)),
            pl.BlockSpec((1, d_out), lambda i: (0, 0)),
        ],
        out_specs=pl.BlockSpec((_BM, d_out), lambda i: (i, 0)),
        out_shape=jax.ShapeDtypeStruct((n, d_out), jnp.float32),
        scratch_shapes=[pltpu.VMEM((n, d_out), jnp.float32)],
        compiler_params=pltpu.CompilerParams(
            vmem_limit_bytes=64 * 1024 * 1024,
        ),
    )(adj, x, W, b.reshape(1, d_out))
    return out.reshape(1, n, d_out)


# speedup vs baseline: 1.0055x; 1.0055x over previous
import jax
import jax.numpy as jnp
from jax.experimental import pallas as pl
from jax.experimental.pallas import tpu as pltpu


_BM = 400


def _gcn_kernel(adj_ref, x_ref, w_ref, b_ref, out_ref, h_ref):
    @pl.when(pl.program_id(0) == 0)
    def _compute_h():
        h_ref[...] = jnp.dot(
            x_ref[...], w_ref[...], preferred_element_type=jnp.float32
        )

    out_ref[...] = (
        jnp.dot(adj_ref[...], h_ref[...], preferred_element_type=jnp.float32)
        + b_ref[...]
    )


def kernel(x, adj, W, b):
    n, d_in = x.shape
    d_out = W.shape[1]
    out = pl.pallas_call(
        _gcn_kernel,
        grid=(n // _BM,),
        in_specs=[
            pl.BlockSpec((_BM, n), lambda i: (i, 0)),
            pl.BlockSpec((n, d_in), lambda i: (0, 0)),
            pl.BlockSpec((d_in, d_out), lambda i: (0, 0)),
            pl.BlockSpec((1, d_out), lambda i: (0, 0)),
        ],
        out_specs=pl.BlockSpec((_BM, d_out), lambda i: (i, 0)),
        out_shape=jax.ShapeDtypeStruct((n, d_out), jnp.float32),
        scratch_shapes=[pltpu.VMEM((n, d_out), jnp.float32)],
        compiler_params=pltpu.CompilerParams(
            vmem_limit_bytes=64 * 1024 * 1024,
        ),
    )(adj, x, W, b.reshape(1, d_out))
    return out.reshape(1, n, d_out)
